# Initial kernel scaffold; baseline (speedup 1.0000x reference)
import jax
import jax.numpy as jnp
from jax.experimental import pallas as pl


def kernel(x):
    raise NotImplementedError("write your pallas kernel here")


# speedup vs baseline: 1.1838x; 1.1838x over previous
import functools

import jax
import jax.numpy as jnp
from jax.experimental import pallas as pl
from jax.experimental.pallas import tpu as pltpu

_NEG = -3.0e38


def _prefix_sum_lanes(v, lane):
    n = v.shape[1]
    sh = 1
    while sh < n:
        v = v + jnp.where(lane >= sh, pltpu.roll(v, sh, axis=1), 0.0)
        sh *= 2
    return v


def _topk_membership(s, k):
    rows = s.shape[0]
    zero_i = jnp.zeros((rows, 1), jnp.int32)

    def body(carry):
        live, r, thr, need, done = carry
        m = jnp.max(live, axis=1, keepdims=True)
        eq = live == m
        c = jnp.sum(eq.astype(jnp.int32), axis=1, keepdims=True)
        newdone = jnp.logical_and(jnp.logical_not(done), (r + c) >= k)
        thr = jnp.where(newdone, m, thr)
        need = jnp.where(newdone, k - r, need)
        cont = jnp.logical_and(jnp.logical_not(done), jnp.logical_not(newdone))
        live = jnp.where(jnp.logical_and(eq, cont), _NEG, live)
        r = jnp.where(cont, r + c, r)
        done = jnp.logical_or(done, newdone)
        return live, r, thr, need, done

    carry = (s, zero_i, jnp.full((rows, 1), _NEG), zero_i,
             jnp.zeros((rows, 1), jnp.bool_))
    for _ in range(k):
        carry = body(carry)
    _, _, thr, need, _ = carry

    lane = jax.lax.broadcasted_iota(jnp.int32, s.shape, 1)
    eq_t = s == thr
    prefix = _prefix_sum_lanes(eq_t.astype(jnp.float32), lane)
    member = jnp.logical_or(s > thr,
                            jnp.logical_and(eq_t, prefix <= need.astype(jnp.float32)))
    return member


def _body(x_ref, out_ref):
    score = x_ref[:, :, 0, 1:]
    b, h, n = score.shape
    s = score.reshape(b * h, n)

    member = _topk_membership(s, 24)
    count = member.astype(jnp.float32).reshape(b, h, n).sum(axis=1)

    lane = jax.lax.broadcasted_iota(jnp.int32, (b, n), 1)
    xc = lane % 24
    yc = lane // 24
    hpass = 2.0 * count
    hpass += jnp.where(xc > 0, pltpu.roll(count, 1, axis=1), 0.0)
    hpass += jnp.where(xc < 23, pltpu.roll(count, n - 1, axis=1), 0.0)
    e = 2.0 * hpass
    e += jnp.where(yc > 0, pltpu.roll(hpass, 24, axis=1), 0.0)
    e += jnp.where(yc < 23, pltpu.roll(hpass, n - 24, axis=1), 0.0)

    for t in range(24):
        m = jnp.max(e, axis=1, keepdims=True)
        idx = jnp.min(jnp.where(e == m, lane, n), axis=1, keepdims=True)
        out_ref[:, t:t + 1] = idx + 1
        e = jnp.where(lane == idx, _NEG, e)


@jax.jit
def kernel(x):
    b, h, m, _ = x.shape
    return pl.pallas_call(
        _body,
        grid=(1,),
        in_specs=[pl.BlockSpec((b, h, 8, m), lambda i: (0, 0, 0, 0))],
        out_specs=pl.BlockSpec((b, 24), lambda i: (0, 0)),
        out_shape=jax.ShapeDtypeStruct((b, 24), jnp.int32),
    )(x)


# speedup vs baseline: 2.7085x; 2.2880x over previous
import jax
import jax.numpy as jnp
from jax import lax
from jax.experimental import pallas as pl
from jax.experimental.pallas import tpu as pltpu
from jax.experimental.pallas import tpu_sc as plsc

L = 16
N = 576
NCHUNK = N // L
K = 24
NEG = -3.0e38


def _iota():
    return lax.iota(jnp.int32, L)


def _splat_max_f(v, ref):
    io = _iota()
    for sh in (1, 2, 4, 8):
        ref[pl.ds(0, L)] = v
        g = plsc.load_gather(ref, [jnp.bitwise_xor(io, sh)])
        v = jnp.maximum(v, g)
    return v


def _splat_sum_i(v, ref):
    io = _iota()
    for sh in (1, 2, 4, 8):
        ref[pl.ds(0, L)] = v
        g = plsc.load_gather(ref, [jnp.bitwise_xor(io, sh)])
        v = v + g
    return v


def _splat_min_i(v, ref):
    io = _iota()
    for sh in (1, 2, 4, 8):
        ref[pl.ds(0, L)] = v
        g = plsc.load_gather(ref, [jnp.bitwise_xor(io, sh)])
        v = jnp.minimum(v, g)
    return v


def _prefix_incl_i(v, ref):
    io = _iota()
    for sh in (1, 2, 4, 8):
        ref[pl.ds(0, L)] = v
        g = plsc.load_gather(ref, [jnp.maximum(io - sh, 0)])
        v = v + jnp.where(io >= sh, g, 0)
    return v


def _sc_body(score_hbm, out_hbm, rows_v, live_v, cnt_v, tot_v, h_v, e_v,
             out_v, four_v, redf_v, redi_v, shared):
    c = lax.axis_index("c")
    s = lax.axis_index("s")
    b = 4 * c + s // 4
    hg = s % 4
    r0 = b * 12 + 3 * hg
    pltpu.sync_copy(score_hbm.at[pl.ds(r0 * N, 3 * N)], rows_v)

    io = _iota()
    zero16 = jnp.zeros((L,), jnp.float32)

    def zero_cnt(j, _):
        cnt_v[pl.ds(pl.multiple_of(j * L, L), L)] = zero16
        return 0

    lax.fori_loop(0, NCHUNK, zero_cnt, 0)

    for rr in range(3):
        def maxpass(j, m):
            base = pl.multiple_of(j * L, L)
            v = rows_v[pl.ds(rr * N + base, L)]
            live_v[pl.ds(base, L)] = v
            return jnp.maximum(m, v)

        m0 = lax.fori_loop(0, NCHUNK, maxpass,
                           jnp.full((L,), NEG, jnp.float32))

        def it(_, carry):
            mvec, r, thr, need, done = carry
            ms = _splat_max_f(mvec, redf_v)

            def sweep(j, st):
                cntv, nm = st
                base = pl.multiple_of(j * L, L)
                v = live_v[pl.ds(base, L)]
                eq = v == ms
                cntv = cntv + jnp.where(eq, 1, 0)
                vn = jnp.where(eq, NEG, v)
                live_v[pl.ds(base, L)] = vn
                return cntv, jnp.maximum(nm, vn)

            cntv, nm = lax.fori_loop(
                0, NCHUNK, sweep,
                (jnp.zeros((L,), jnp.int32),
                 jnp.full((L,), NEG, jnp.float32)))
            cc = _splat_sum_i(cntv, redi_v)
            newdone = jnp.where(
                jnp.logical_and(done == 0, (r + cc) >= K), 1, 0)
            thr = jnp.where(newdone > 0, ms, thr)
            need = jnp.where(newdone > 0, K - r, need)
            r = jnp.where(done > 0, r, r + cc)
            done = jnp.maximum(done, newdone)
            return nm, r, thr, need, done

        zero_i = jnp.zeros((L,), jnp.int32)
        carry = (m0, zero_i, jnp.full((L,), NEG, jnp.float32), zero_i,
                 zero_i)
        _, _, thr, need, _ = lax.fori_loop(0, K, it, carry)

        def member(j, run):
            base = pl.multiple_of(j * L, L)
            v = rows_v[pl.ds(rr * N + base, L)]
            gt = v > thr
            eqt = v == thr
            eqi = jnp.where(eqt, 1, 0)
            pre = _prefix_incl_i(eqi, redi_v)
            redi_v[pl.ds(0, L)] = pre
            tot = plsc.load_gather(redi_v, [jnp.full((L,), L - 1)])
            elig = jnp.logical_and(eqt, (run + pre) <= need)
            memb = jnp.where(jnp.logical_or(gt, elig), 1.0, 0.0)
            cnt_v[pl.ds(base, L)] = cnt_v[pl.ds(base, L)] + memb
            return run + tot

        lax.fori_loop(0, NCHUNK, member, jnp.zeros((L,), jnp.int32))

    pltpu.sync_copy(cnt_v, shared.at[pl.ds(s * N, N)])
    plsc.subcore_barrier()

    @pl.when(s % 4 == 0)
    def _leader():
        pltpu.sync_copy(shared.at[pl.ds(s * N, 4 * N)], four_v)

        def sum4(j, _):
            base = pl.multiple_of(j * L, L)
            t0 = four_v[pl.ds(base, L)] + four_v[pl.ds(N + base, L)]
            t1 = (four_v[pl.ds(2 * N + base, L)]
                  + four_v[pl.ds(3 * N + base, L)])
            tot_v[pl.ds(base, L)] = t0 + t1
            return 0

        lax.fori_loop(0, NCHUNK, sum4, 0)

        def hconv(j, _):
            base = pl.multiple_of(j * L, L)
            p = io + base
            xc = p % 24
            t = tot_v[pl.ds(base, L)]
            left = plsc.load_gather(tot_v, [jnp.maximum(p - 1, 0)])
            right = plsc.load_gather(tot_v, [jnp.minimum(p + 1, N - 1)])
            h_v[pl.ds(base, L)] = (2.0 * t
                                   + jnp.where(xc > 0, left, 0.0)
                                   + jnp.where(xc < 23, right, 0.0))
            return 0

        lax.fori_loop(0, NCHUNK, hconv, 0)

        def vconv(j, _):
            base = pl.multiple_of(j * L, L)
            p = io + base
            t = h_v[pl.ds(base, L)]
            up = plsc.load_gather(h_v, [jnp.maximum(p - 24, 0)])
            dn = plsc.load_gather(h_v, [jnp.minimum(p + 24, N - 1)])
            e_v[pl.ds(base, L)] = (2.0 * t
                                   + jnp.where(p >= 24, up, 0.0)
                                   + jnp.where(p < N - 24, dn, 0.0))
            return 0

        lax.fori_loop(0, NCHUNK, vconv, 0)

        def ext(t, carry):
            o0, o1 = carry

            def maxp(j, m):
                return jnp.maximum(
                    m, e_v[pl.ds(pl.multiple_of(j * L, L), L)])

            m = lax.fori_loop(0, NCHUNK, maxp,
                              jnp.full((L,), NEG, jnp.float32))
            ms = _splat_max_f(m, redf_v)

            def idxp(j, bidx):
                base = pl.multiple_of(j * L, L)
                v = e_v[pl.ds(base, L)]
                return jnp.minimum(bidx,
                                   jnp.where(v == ms, io + base, N))

            bidx = lax.fori_loop(0, NCHUNK, idxp,
                                 jnp.full((L,), N, jnp.int32))
            ai = _splat_min_i(bidx, redi_v)
            plsc.store_scatter(e_v, [ai], jnp.full((L,), NEG, jnp.float32),
                               mask=io == 0)
            tv = jnp.full((L,), t, jnp.int32)
            val = (ai + 1).astype(jnp.int32)
            o0 = jnp.where(jnp.logical_and(tv < L, io == tv), val, o0)
            o1 = jnp.where(jnp.logical_and(tv >= L, io == tv - L), val, o1)
            return o0, o1

        zero_i = jnp.zeros((L,), jnp.int32)
        o0, o1 = lax.fori_loop(0, K, ext, (zero_i, zero_i))
        out_v[pl.ds(0, L)] = o0
        out_v[pl.ds(L, L)] = o1
        pltpu.sync_copy(out_v, out_hbm.at[pl.ds(b * 32, 32)])


@jax.jit
def kernel(x):
    bb, hh, mm, _ = x.shape
    score = x[:, :, 0, 1:].reshape(bb * hh * (mm - 1))
    mesh = plsc.VectorSubcoreMesh(core_axis_name="c", subcore_axis_name="s")
    run = pl.kernel(
        _sc_body,
        mesh=mesh,
        compiler_params=pltpu.CompilerParams(needs_layout_passes=False),
        out_type=jax.ShapeDtypeStruct((bb * 32,), jnp.int32),
        scratch_types=[
            pltpu.VMEM((3 * N,), jnp.float32),
            pltpu.VMEM((N,), jnp.float32),
            pltpu.VMEM((N,), jnp.float32),
            pltpu.VMEM((N,), jnp.float32),
            pltpu.VMEM((N,), jnp.float32),
            pltpu.VMEM((N,), jnp.float32),
            pltpu.VMEM((32,), jnp.int32),
            pltpu.VMEM((4 * N,), jnp.float32),
            pltpu.VMEM((L,), jnp.float32),
            pltpu.VMEM((L,), jnp.int32),
            pltpu.VMEM_SHARED((16 * N,), jnp.float32),
        ],
    )
    return run(score).reshape(bb, 32)[:, :K]


# speedup vs baseline: 4.1265x; 1.5236x over previous
import jax
import jax.numpy as jnp
from jax import lax
from jax.experimental import pallas as pl
from jax.experimental.pallas import tpu as pltpu
from jax.experimental.pallas import tpu_sc as plsc

L = 16
N = 576
NCHUNK = N // L
K = 24
NEG = -3.0e38
BIG = 4096


def _iota():
    return lax.iota(jnp.int32, L)


def _splat_sum_i(v, ref):
    io = _iota()
    for sh in (1, 2, 4, 8):
        ref[pl.ds(0, L)] = v
        g = plsc.load_gather(ref, [jnp.bitwise_xor(io, sh)])
        v = v + g
    return v


def _splat_min_i(v, ref):
    io = _iota()
    for sh in (1, 2, 4, 8):
        ref[pl.ds(0, L)] = v
        g = plsc.load_gather(ref, [jnp.bitwise_xor(io, sh)])
        v = jnp.minimum(v, g)
    return v


def _sortd(v, descending=True):
    k, _ = plsc.sort_key_val(v, v, descending=descending)
    return k


def _merge16(s1, s2):
    r = jnp.flip(s2)
    hi = jnp.maximum(s1, r)
    lo = jnp.minimum(s1, r)
    return _sortd(hi), _sortd(lo)


def _merge32(x, y):
    x1, x2 = x
    y1, y2 = y
    t1 = jnp.maximum(x1, jnp.flip(y2))
    t2 = jnp.maximum(x2, jnp.flip(y1))
    return _merge16(_sortd(t1), _sortd(t2))


def _top32_tree(chunks):
    sorted_chunks = [_sortd(c) for c in chunks]
    nodes = [_merge16(sorted_chunks[2 * i], sorted_chunks[2 * i + 1])
             for i in range(len(sorted_chunks) // 2)]
    if len(sorted_chunks) % 2:
        s = sorted_chunks[-1]
        pad = jnp.full((L,), s.dtype.type(0), s.dtype)
        nodes.append((s, pad))
    while len(nodes) > 1:
        nxt = [_merge32(nodes[2 * i], nodes[2 * i + 1])
               for i in range(len(nodes) // 2)]
        if len(nodes) % 2:
            nxt.append(nodes[-1])
        nodes = nxt
    return nodes[0]


def _sc_body(score_hbm, out_hbm, rows_v, cnt_v, tot_v, h_v, e_v,
             out_v, four_v, redi_v, shared):
    c = lax.axis_index("c")
    s = lax.axis_index("s")
    b = 4 * c + s // 4
    hg = s % 4
    r0 = b * 12 + 3 * hg
    pltpu.sync_copy(score_hbm.at[pl.ds(r0 * N, 3 * N)], rows_v)

    io = _iota()
    zero16 = jnp.zeros((L,), jnp.float32)
    for j in range(NCHUNK):
        cnt_v[pl.ds(j * L, L)] = zero16

    for rr in range(3):
        chunks = [rows_v[pl.ds(rr * N + j * L, L)] for j in range(NCHUNK)]
        r1, r2 = _top32_tree(chunks)
        thr = r2[K - L - 1]
        tsp = jnp.full((L,), thr, jnp.float32)

        gcnt = (jnp.where(r1 > tsp, 1, 0) + jnp.where(r2 > tsp, 1, 0))
        need = K - _splat_sum_i(gcnt, redi_v)[0]

        def fcond(st):
            i, _ = st
            return i < need

        def fbody(st):
            i, cc = st
            csp = jnp.full((L,), cc, jnp.int32)
            bidx = jnp.full((L,), BIG, jnp.int32)
            for j in range(NCHUNK):
                v = rows_v[pl.ds(rr * N + j * L, L)]
                p = io + j * L
                hit = jnp.logical_and(v == tsp, p > csp)
                bidx = jnp.minimum(bidx, jnp.where(hit, p, BIG))
            return i + 1, _splat_min_i(bidx, redi_v)[0]

        _, cut = lax.while_loop(fcond, fbody, (jnp.int32(0), jnp.int32(-1)))
        csp = jnp.full((L,), cut, jnp.int32)

        for j in range(NCHUNK):
            v = rows_v[pl.ds(rr * N + j * L, L)]
            p = io + j * L
            memb = jnp.logical_or(
                v > tsp, jnp.logical_and(v == tsp, p <= csp))
            cnt_v[pl.ds(j * L, L)] = (cnt_v[pl.ds(j * L, L)]
                                      + jnp.where(memb, 1.0, 0.0))

    pltpu.sync_copy(cnt_v, shared.at[pl.ds(s * N, N)])
    plsc.subcore_barrier()

    @pl.when(s % 4 == 0)
    def _leader():
        pltpu.sync_copy(shared.at[pl.ds(s * N, 4 * N)], four_v)
        for j in range(NCHUNK):
            base = j * L
            t0 = four_v[pl.ds(base, L)] + four_v[pl.ds(N + base, L)]
            t1 = (four_v[pl.ds(2 * N + base, L)]
                  + four_v[pl.ds(3 * N + base, L)])
            tot_v[pl.ds(base, L)] = t0 + t1

        for j in range(NCHUNK):
            base = j * L
            p = io + base
            xc = p % 24
            t = tot_v[pl.ds(base, L)]
            left = plsc.load_gather(tot_v, [jnp.maximum(p - 1, 0)])
            right = plsc.load_gather(tot_v, [jnp.minimum(p + 1, N - 1)])
            h_v[pl.ds(base, L)] = (2.0 * t
                                   + jnp.where(xc > 0, left, 0.0)
                                   + jnp.where(xc < 23, right, 0.0))

        for j in range(NCHUNK):
            base = j * L
            p = io + base
            t = h_v[pl.ds(base, L)]
            up = plsc.load_gather(h_v, [jnp.maximum(p - 24, 0)])
            dn = plsc.load_gather(h_v, [jnp.minimum(p + 24, N - 1)])
            e_v[pl.ds(base, L)] = (2.0 * t
                                   + jnp.where(p >= 24, up, 0.0)
                                   + jnp.where(p < N - 24, dn, 0.0))

        keys = []
        for j in range(NCHUNK):
            base = j * L
            p = io + base
            ei = e_v[pl.ds(base, L)].astype(jnp.int32)
            keys.append(jnp.bitwise_or(jnp.left_shift(ei, 10), 1023 - p))
        k1, k2 = _top32_tree(keys)
        o0 = 1024 - jnp.bitwise_and(k1, 1023)
        o1 = 1024 - jnp.bitwise_and(k2, 1023)
        out_v[pl.ds(0, L)] = o0
        out_v[pl.ds(L, L)] = o1
        pltpu.sync_copy(out_v, out_hbm.at[pl.ds(b * 32, 32)])


@jax.jit
def kernel(x):
    bb, hh, mm, _ = x.shape
    score = x[:, :, 0, 1:].reshape(bb * hh * (mm - 1))
    mesh = plsc.VectorSubcoreMesh(core_axis_name="c", subcore_axis_name="s")
    run = pl.kernel(
        _sc_body,
        mesh=mesh,
        compiler_params=pltpu.CompilerParams(needs_layout_passes=False),
        out_type=jax.ShapeDtypeStruct((bb * 32,), jnp.int32),
        scratch_types=[
            pltpu.VMEM((3 * N,), jnp.float32),
            pltpu.VMEM((N,), jnp.float32),
            pltpu.VMEM((N,), jnp.float32),
            pltpu.VMEM((N,), jnp.float32),
            pltpu.VMEM((N,), jnp.float32),
            pltpu.VMEM((32,), jnp.int32),
            pltpu.VMEM((4 * N,), jnp.float32),
            pltpu.VMEM((L,), jnp.int32),
            pltpu.VMEM_SHARED((16 * N,), jnp.float32),
        ],
    )
    return run(score).reshape(bb, 32)[:, :K]
